# baseline (speedup 1.0000x reference)
import jax
import jax.numpy as jnp
from jax.experimental import pallas as pl
from jax.experimental.pallas import tpu as pltpu

B = 16384
D = 64
DC = 16
ND = D // DC


def _body(gu_ref, gi_ref, o_ref):
    i = pl.program_id(0)
    part = jnp.sum(gu_ref[...] * gi_ref[...], axis=0)

    @pl.when(i == 0)
    def _():
        o_ref[...] = part

    @pl.when(i > 0)
    def _():
        o_ref[...] += part


_call = pl.pallas_call(
    _body,
    grid=(ND,),
    in_specs=[
        pl.BlockSpec((DC, B), lambda i: (i, 0)),
        pl.BlockSpec((DC, B), lambda i: (i, 0)),
    ],
    out_specs=pl.BlockSpec((B,), lambda i: (0,)),
    out_shape=jax.ShapeDtypeStruct((B,), jnp.float32),
    compiler_params=pltpu.CompilerParams(
        dimension_semantics=("arbitrary",),
    ),
)


@jax.jit
def kernel(gu, gi):
    gut = pltpu.with_memory_space_constraint(gu.T, pltpu.MemorySpace.HBM)
    git = pltpu.with_memory_space_constraint(gi.T, pltpu.MemorySpace.HBM)
    return _call(gut, git)


# speedup vs baseline: 1.1256x; 1.1256x over previous
import jax
import jax.numpy as jnp
from jax.experimental import pallas as pl
from jax.experimental.pallas import tpu as pltpu

B = 16384
D = 64
DC = 32
ND = D // DC


def _body(gu_ref, gi_ref, o_ref):
    i = pl.program_id(0)
    part = jnp.sum(gu_ref[...] * gi_ref[...], axis=0)

    @pl.when(i == 0)
    def _():
        o_ref[...] = part

    @pl.when(i > 0)
    def _():
        o_ref[...] += part


_call = pl.pallas_call(
    _body,
    grid=(ND,),
    in_specs=[
        pl.BlockSpec((DC, B), lambda i: (i, 0)),
        pl.BlockSpec((DC, B), lambda i: (i, 0)),
    ],
    out_specs=pl.BlockSpec((B,), lambda i: (0,)),
    out_shape=jax.ShapeDtypeStruct((B,), jnp.float32),
    compiler_params=pltpu.CompilerParams(
        dimension_semantics=("arbitrary",),
    ),
)


@jax.jit
def kernel(gu, gi):
    gut = pltpu.with_memory_space_constraint(gu.T, pltpu.MemorySpace.HBM)
    git = pltpu.with_memory_space_constraint(gi.T, pltpu.MemorySpace.HBM)
    return _call(gut, git)


# speedup vs baseline: 1.1534x; 1.0247x over previous
import jax
import jax.numpy as jnp
from jax.experimental import pallas as pl
from jax.experimental.pallas import tpu as pltpu

B = 16384
D = 64
BLK = 8192
NB = B // BLK


def _body(gu_ref, gi_ref, o_ref):
    o_ref[...] = jnp.sum(gu_ref[...] * gi_ref[...], axis=0)


_call = pl.pallas_call(
    _body,
    grid=(NB,),
    in_specs=[
        pl.BlockSpec((D, BLK), lambda i: (0, i)),
        pl.BlockSpec((D, BLK), lambda i: (0, i)),
    ],
    out_specs=pl.BlockSpec((BLK,), lambda i: (i,)),
    out_shape=jax.ShapeDtypeStruct((B,), jnp.float32),
    compiler_params=pltpu.CompilerParams(
        dimension_semantics=("parallel",),
    ),
)


@jax.jit
def kernel(gu, gi):
    gut = pltpu.with_memory_space_constraint(gu.T, pltpu.MemorySpace.HBM)
    git = pltpu.with_memory_space_constraint(gi.T, pltpu.MemorySpace.HBM)
    return _call(gut, git)


# speedup vs baseline: 1.1549x; 1.0014x over previous
import jax
import jax.numpy as jnp
from jax.experimental import pallas as pl
from jax.experimental.pallas import tpu as pltpu

B = 16384
D = 64
BLK = 8192
NB = B // BLK


def _body(gu_ref, gi_ref, o_ref):
    o_ref[...] = jnp.sum(gu_ref[...] * gi_ref[...], axis=0)


_call = pl.pallas_call(
    _body,
    grid=(NB,),
    in_specs=[
        pl.BlockSpec((D, BLK), lambda i: (0, i)),
        pl.BlockSpec((D, BLK), lambda i: (0, i)),
    ],
    out_specs=pl.BlockSpec((BLK,), lambda i: (i,)),
    out_shape=jax.ShapeDtypeStruct((B,), jnp.float32),
    compiler_params=pltpu.CompilerParams(
        dimension_semantics=("parallel",),
        disable_bounds_checks=True,
        skip_device_barrier=True,
    ),
)


@jax.jit
def kernel(gu, gi):
    gut = pltpu.with_memory_space_constraint(gu.T, pltpu.MemorySpace.HBM)
    git = pltpu.with_memory_space_constraint(gi.T, pltpu.MemorySpace.HBM)
    return _call(gut, git)
